# Initial kernel scaffold; baseline (speedup 1.0000x reference)
import jax
import jax.numpy as jnp
from jax.experimental import pallas as pl


def kernel(hidden_states, gate_w, w_gate, w_up, w_down):
    raise NotImplementedError("write your pallas kernel here")


# speedup vs baseline: 1.2194x; 1.2194x over previous
import functools

import jax
import jax.numpy as jnp
from jax.experimental import pallas as pl
from jax.experimental.pallas import tpu as pltpu

NUM_EXPERTS = 8
TOP_K = 2
D_MODEL = 1024
D_FF = 1408
NUM_TOKENS = 2048

BM = 256
NB = (NUM_TOKENS * TOP_K) // BM + (NUM_EXPERTS - 1)
P_ROWS = NB * BM


def _moe_block_body(be_ref, xs_ref, wg_ref, wu_ref, wd_ref, wt_ref, ys_ref):
    x = xs_ref[...]
    wg = wg_ref[0]
    wu = wu_ref[0]
    wd = wd_ref[0]
    g = jax.lax.dot_general(x, wg, (((1,), (1,)), ((), ())),
                            preferred_element_type=jnp.float32)
    u = jax.lax.dot_general(x, wu, (((1,), (1,)), ((), ())),
                            preferred_element_type=jnp.float32)
    h = g * jax.nn.sigmoid(g) * u
    y = jax.lax.dot_general(h, wd, (((1,), (1,)), ((), ())),
                            preferred_element_type=jnp.float32)
    wt = wt_ref[0, 0]
    ys_ref[...] = y * wt[:, None]


def _grouped_mlp(blk_exp, xs, w_gate, w_up, w_down, wt3):
    grid_spec = pltpu.PrefetchScalarGridSpec(
        num_scalar_prefetch=1,
        grid=(NB,),
        in_specs=[
            pl.BlockSpec((BM, D_MODEL), lambda b, be: (b, 0)),
            pl.BlockSpec((1, D_FF, D_MODEL), lambda b, be: (be[b], 0, 0)),
            pl.BlockSpec((1, D_FF, D_MODEL), lambda b, be: (be[b], 0, 0)),
            pl.BlockSpec((1, D_MODEL, D_FF), lambda b, be: (be[b], 0, 0)),
            pl.BlockSpec((1, 1, BM), lambda b, be: (b, 0, 0)),
        ],
        out_specs=pl.BlockSpec((BM, D_MODEL), lambda b, be: (b, 0)),
    )
    return pl.pallas_call(
        _moe_block_body,
        grid_spec=grid_spec,
        out_shape=jax.ShapeDtypeStruct((P_ROWS, D_MODEL), jnp.float32),
    )(blk_exp, xs, w_gate, w_up, w_down, wt3)


def kernel(hidden_states, gate_w, w_gate, w_up, w_down):
    T, D = hidden_states.shape
    router_logits = hidden_states @ gate_w.T
    probs = jax.nn.softmax(router_logits.astype(jnp.float32), -1)
    topk_vals, topk_ids = jax.lax.top_k(probs, TOP_K)
    topk_w = topk_vals / jnp.sum(topk_vals, axis=-1, keepdims=True)

    flat_e = topk_ids.T.reshape(-1)
    flat_w = topk_w.T.reshape(-1).astype(jnp.float32)
    onehot = (flat_e[:, None] == jnp.arange(NUM_EXPERTS)[None, :]).astype(jnp.int32)
    cum = jnp.cumsum(onehot, axis=0)
    rank = jnp.take_along_axis(cum, flat_e[:, None], axis=1)[:, 0] - 1
    counts = cum[-1]
    nblk = (counts + BM - 1) // BM
    blk_start = jnp.concatenate([jnp.zeros((1,), jnp.int32),
                                 jnp.cumsum(nblk)[:-1].astype(jnp.int32)])
    dest = blk_start[flat_e] * BM + rank
    tok = jnp.zeros((P_ROWS,), jnp.int32).at[dest].set(
        jnp.arange(T * TOP_K, dtype=jnp.int32) % T)
    wt = jnp.zeros((P_ROWS,), jnp.float32).at[dest].set(flat_w)
    blk_exp = jnp.clip(
        jnp.searchsorted(jnp.cumsum(nblk), jnp.arange(NB), side="right"),
        0, NUM_EXPERTS - 1).astype(jnp.int32)

    xs = jnp.take(hidden_states, tok, axis=0)
    wt3 = wt.reshape(NB, 1, BM)
    ys = _grouped_mlp(blk_exp, xs, w_gate, w_up, w_down, wt3)
    out = jnp.take(ys, dest[:T], axis=0) + jnp.take(ys, dest[T:], axis=0)
    return out


# speedup vs baseline: 1.6492x; 1.3525x over previous
import functools

import jax
import jax.numpy as jnp
from jax import lax
from jax.experimental import pallas as pl
from jax.experimental.pallas import tpu as pltpu
from jax.experimental.pallas import tpu_sc as plsc

NUM_EXPERTS = 8
TOP_K = 2
D_MODEL = 1024
D_FF = 1408
NUM_TOKENS = 2048

BM = 256
NB = (NUM_TOKENS * TOP_K) // BM + (NUM_EXPERTS - 1)
P_ROWS = NB * BM

_SC_NC, _SC_NS = 2, 16
_NW = _SC_NC * _SC_NS
_SLOTS_W = NUM_TOKENS * TOP_K // _NW
_TOK_W = NUM_TOKENS // _NW


def _sc_mesh():
    return plsc.VectorSubcoreMesh(core_axis_name="c", subcore_axis_name="s",
                                  num_cores=_SC_NC, num_subcores=_SC_NS)


_DCH = 4
_DCR = _SLOTS_W // _DCH


def _dispatch_body(x_hbm, dest_hbm, w_hbm, xs_hbm, wt_hbm,
                   idx_v, w_v, rows0, rows1, sem_r0, sem_r1,
                   sem_s0, sem_s1, sem_w):
    wid = lax.axis_index("s") * _SC_NC + lax.axis_index("c")
    tbase = lax.rem(wid, _NW // TOP_K) * _SLOTS_W
    s0 = wid * _SLOTS_W
    for c in range(_DCH):
        pltpu.sync_copy(dest_hbm.at[pl.ds(s0 + c * _DCR, _DCR)], idx_v.at[c])
        pltpu.sync_copy(w_hbm.at[pl.ds(s0 + c * _DCR, _DCR)], w_v.at[c])
    bufs = (rows0, rows1)
    rsem = (sem_r0, sem_r1)
    ssem = (sem_s0, sem_s1)

    def rd(c):
        return pltpu.async_copy(
            x_hbm.at[pl.ds(tbase + c * _DCR, _DCR)], bufs[c % 2], rsem[c % 2])

    def sc(c):
        return pltpu.async_copy(bufs[c % 2], xs_hbm.at[idx_v.at[c]], ssem[c % 2])

    rds = [None] * _DCH
    scs = [None] * _DCH
    wss = [None] * _DCH
    rds[0] = rd(0)
    rds[1] = rd(1)
    for c in range(_DCH):
        rds[c].wait()
        scs[c] = sc(c)
        wss[c] = pltpu.async_copy(w_v.at[c], wt_hbm.at[idx_v.at[c]], sem_w)
        if c + 2 < _DCH:
            scs[c].wait()
            rds[c + 2] = rd(c + 2)
    scs[_DCH - 2].wait()
    scs[_DCH - 1].wait()
    for c in range(_DCH):
        wss[c].wait()


def _sc_dispatch(hidden_states, dest, wt_flat):
    return pl.kernel(
        _dispatch_body,
        out_type=(jax.ShapeDtypeStruct((P_ROWS, D_MODEL), jnp.float32),
                  jax.ShapeDtypeStruct((P_ROWS,), jnp.float32)),
        mesh=_sc_mesh(),
        scratch_types=[
            pltpu.VMEM((_DCH, _DCR), jnp.int32),
            pltpu.VMEM((_DCH, _DCR), jnp.float32),
            pltpu.VMEM((_DCR, D_MODEL), jnp.float32),
            pltpu.VMEM((_DCR, D_MODEL), jnp.float32),
            pltpu.SemaphoreType.DMA,
            pltpu.SemaphoreType.DMA,
            pltpu.SemaphoreType.DMA,
            pltpu.SemaphoreType.DMA,
            pltpu.SemaphoreType.DMA,
        ],
    )(hidden_states, dest, wt_flat)


_CCH = 4
_CCR = _TOK_W // _CCH


def _combine_body(ys_hbm, dest_hbm, out_hbm, i0, i1,
                  r0a, r1a, r0b, r1b, g0s, g1s, sts):
    wid = lax.axis_index("s") * _SC_NC + lax.axis_index("c")
    t0 = wid * _TOK_W
    for c in range(_CCH):
        pltpu.sync_copy(dest_hbm.at[pl.ds(t0 + c * _CCR, _CCR)], i0.at[c])
        pltpu.sync_copy(dest_hbm.at[pl.ds(NUM_TOKENS + t0 + c * _CCR, _CCR)],
                        i1.at[c])
    b0 = (r0a, r0b)
    b1 = (r1a, r1b)

    def gath(c):
        p = c % 2
        return (pltpu.async_copy(ys_hbm.at[i0.at[c]], b0[p], g0s),
                pltpu.async_copy(ys_hbm.at[i1.at[c]], b1[p], g1s))

    gs = [None] * _CCH
    st = [None] * _CCH
    gs[0] = gath(0)
    gs[1] = gath(1)
    for c in range(_CCH):
        p = c % 2
        gs[c][0].wait()
        gs[c][1].wait()
        ra, rb = b0[p], b1[p]

        def row_loop(i, carry):
            def col_loop(j, carry2):
                sl = pl.ds(j * 16, 16)
                ra[i, sl] = ra[i, sl] + rb[i, sl]
                return carry2
            return lax.fori_loop(0, D_MODEL // 16, col_loop, carry)

        lax.fori_loop(0, _CCR, row_loop, 0)
        st[c] = pltpu.async_copy(ra, out_hbm.at[pl.ds(t0 + c * _CCR, _CCR)], sts)
        if c + 2 < _CCH:
            st[c].wait()
            gs[c + 2] = gath(c + 2)
    st[_CCH - 2].wait()
    st[_CCH - 1].wait()


def _sc_combine(ys, dest):
    return pl.kernel(
        _combine_body,
        out_type=jax.ShapeDtypeStruct((NUM_TOKENS, D_MODEL), jnp.float32),
        mesh=_sc_mesh(),
        scratch_types=[
            pltpu.VMEM((_CCH, _CCR), jnp.int32),
            pltpu.VMEM((_CCH, _CCR), jnp.int32),
            pltpu.VMEM((_CCR, D_MODEL), jnp.float32),
            pltpu.VMEM((_CCR, D_MODEL), jnp.float32),
            pltpu.VMEM((_CCR, D_MODEL), jnp.float32),
            pltpu.VMEM((_CCR, D_MODEL), jnp.float32),
            pltpu.SemaphoreType.DMA,
            pltpu.SemaphoreType.DMA,
            pltpu.SemaphoreType.DMA,
        ],
    )(ys, dest)


def _moe_block_body(be_ref, pfe_ref, pfdo_ref, slot_ref,
                    xs_ref, wg_any, wu_any, wd_any, wt_ref, ys_ref,
                    wgb, wub, wdb, sem0, sem1):
    b = pl.program_id(0)
    e = be_ref[b]
    slot = slot_ref[b]
    boundary = jnp.logical_or(b == 0, e != be_ref[jnp.maximum(b, 1) - 1])
    pf = pfdo_ref[b] == 1
    pe = pfe_ref[b]

    @pl.when(b == 0)
    def _init():
        pltpu.make_async_copy(wg_any.at[e], wgb.at[0], sem0).start()
        pltpu.make_async_copy(wu_any.at[e], wub.at[0], sem0).start()
        pltpu.make_async_copy(wd_any.at[e], wdb.at[0], sem0).start()

    @pl.when(pf & (slot == 0))
    def _pf_into1():
        pltpu.make_async_copy(wg_any.at[pe], wgb.at[1], sem1).start()
        pltpu.make_async_copy(wu_any.at[pe], wub.at[1], sem1).start()
        pltpu.make_async_copy(wd_any.at[pe], wdb.at[1], sem1).start()

    @pl.when(pf & (slot == 1))
    def _pf_into0():
        pltpu.make_async_copy(wg_any.at[pe], wgb.at[0], sem0).start()
        pltpu.make_async_copy(wu_any.at[pe], wub.at[0], sem0).start()
        pltpu.make_async_copy(wd_any.at[pe], wdb.at[0], sem0).start()

    @pl.when(boundary & (slot == 0))
    def _wait0():
        pltpu.make_async_copy(wg_any.at[e], wgb.at[0], sem0).wait()
        pltpu.make_async_copy(wu_any.at[e], wub.at[0], sem0).wait()
        pltpu.make_async_copy(wd_any.at[e], wdb.at[0], sem0).wait()

    @pl.when(boundary & (slot == 1))
    def _wait1():
        pltpu.make_async_copy(wg_any.at[e], wgb.at[1], sem1).wait()
        pltpu.make_async_copy(wu_any.at[e], wub.at[1], sem1).wait()
        pltpu.make_async_copy(wd_any.at[e], wdb.at[1], sem1).wait()

    x = xs_ref[...]
    wg = wgb[slot]
    wu = wub[slot]
    wd = wdb[slot]
    g = jax.lax.dot_general(x, wg, (((1,), (1,)), ((), ())),
                            preferred_element_type=jnp.float32)
    u = jax.lax.dot_general(x, wu, (((1,), (1,)), ((), ())),
                            preferred_element_type=jnp.float32)
    h = g * jax.nn.sigmoid(g) * u
    y = jax.lax.dot_general(h, wd, (((1,), (1,)), ((), ())),
                            preferred_element_type=jnp.float32)
    ys_ref[...] = y * wt_ref[0, 0][:, None]


def _grouped_mlp(blk_exp, pf_e, pf_do, slot, xs, w_gate, w_up, w_down, wt3):
    grid_spec = pltpu.PrefetchScalarGridSpec(
        num_scalar_prefetch=4,
        grid=(NB,),
        in_specs=[
            pl.BlockSpec((BM, D_MODEL), lambda b, *_: (b, 0)),
            pl.BlockSpec(memory_space=pl.ANY),
            pl.BlockSpec(memory_space=pl.ANY),
            pl.BlockSpec(memory_space=pl.ANY),
            pl.BlockSpec((1, 1, BM), lambda b, *_: (b, 0, 0)),
        ],
        out_specs=pl.BlockSpec((BM, D_MODEL), lambda b, *_: (b, 0)),
        scratch_shapes=[
            pltpu.VMEM((2, D_FF, D_MODEL), jnp.float32),
            pltpu.VMEM((2, D_FF, D_MODEL), jnp.float32),
            pltpu.VMEM((2, D_MODEL, D_FF), jnp.float32),
            pltpu.SemaphoreType.DMA,
            pltpu.SemaphoreType.DMA,
        ],
    )
    return pl.pallas_call(
        _moe_block_body,
        grid_spec=grid_spec,
        out_shape=jax.ShapeDtypeStruct((P_ROWS, D_MODEL), jnp.float32),
    )(blk_exp, pf_e, pf_do, slot, xs, w_gate, w_up, w_down, wt3)


def kernel(hidden_states, gate_w, w_gate, w_up, w_down):
    T, D = hidden_states.shape
    router_logits = hidden_states @ gate_w.T
    probs = jax.nn.softmax(router_logits.astype(jnp.float32), -1)
    topk_vals, topk_ids = jax.lax.top_k(probs, TOP_K)
    topk_w = topk_vals / jnp.sum(topk_vals, axis=-1, keepdims=True)

    flat_e = topk_ids.T.reshape(-1)
    onehot = (flat_e[:, None] == jnp.arange(NUM_EXPERTS)[None, :]).astype(jnp.int32)
    cum = jnp.cumsum(onehot, axis=0)
    counts = cum[-1]
    nblk = (counts + BM - 1) // BM
    cum_nblk = jnp.cumsum(nblk)
    blk_start = cum_nblk - nblk
    dest = jnp.sum(onehot * (blk_start[None, :] * BM + cum - 1), axis=1)
    blk_exp = jnp.minimum(
        jnp.sum((jnp.arange(NB)[:, None] >= cum_nblk[None, :]).astype(jnp.int32),
                axis=1), NUM_EXPERTS - 1).astype(jnp.int32)

    idx = jnp.arange(NB)
    boundary = jnp.concatenate([jnp.array([True]), blk_exp[1:] != blk_exp[:-1]])
    vis = jnp.cumsum(boundary.astype(jnp.int32)) - 1
    slot = (vis % 2).astype(jnp.int32)
    bidx = jnp.where(boundary, idx, NB + 7)
    sufmin = lax.cummin(bidx[::-1])[::-1]
    nxt = jnp.concatenate([sufmin[1:], jnp.array([NB + 7])])
    pf_do = (boundary & (nxt < NB)).astype(jnp.int32)
    nxt_c = jnp.minimum(nxt, NB - 1)
    pf_e = jnp.sum((nxt_c[:, None] == idx[None, :]) * blk_exp[None, :],
                   axis=1).astype(jnp.int32)

    wt_flat = topk_w.T.reshape(-1).astype(jnp.float32)
    xs, wt_sorted = _sc_dispatch(hidden_states, dest, wt_flat)
    wt3 = wt_sorted.reshape(NB, 1, BM)
    ys = _grouped_mlp(blk_exp, pf_e, pf_do, slot, xs,
                      w_gate, w_up, w_down, wt3)
    return _sc_combine(ys, dest)
